# baseline (speedup 1.0000x reference)
import functools

import jax
import jax.numpy as jnp
from jax import lax
from jax.experimental import pallas as pl
from jax.experimental.pallas import tpu as pltpu
from jax.experimental.pallas import tpu_sc as plsc

_EPS = 1e-6
_NC, _NS = 2, 16
_NW = _NC * _NS
_CHUNK = 128


def _sc_gather(users, pos, neg_t, das_c, u_true, u2, v, da_tab):
    B = users.shape[0]
    D = u_true.shape[1]
    NNEG = neg_t.shape[0] // B
    rows_per_w = B // _NW
    n_chunks = rows_per_w // _CHUNK

    mesh = plsc.VectorSubcoreMesh(core_axis_name="c", subcore_axis_name="s")

    @functools.partial(
        pl.kernel,
        out_type=(
            jax.ShapeDtypeStruct((B, D), jnp.float32),
            jax.ShapeDtypeStruct((B, D), jnp.float32),
            jax.ShapeDtypeStruct((B, D), jnp.float32),
            jax.ShapeDtypeStruct((NNEG * B, D), jnp.float32),
        ),
        mesh=mesh,
        scratch_types=(
            pltpu.VMEM((_CHUNK,), jnp.int32),
            pltpu.VMEM((_CHUNK, D), jnp.float32),
            pltpu.VMEM((_CHUNK, 2 * D), jnp.float32),
            pltpu.VMEM((_CHUNK, D), jnp.float32),
            pltpu.VMEM((_CHUNK, D), jnp.float32),
            pltpu.SemaphoreType.DMA,
        ),
        compiler_params=pltpu.CompilerParams(use_tc_tiling_on_sc=False),
    )
    def k(users_h, pos_h, neg_h, das_h, ut_tab, u2_tab, v_tab, da_h,
          up_out, da_out, i_out, j_out,
          idx_v, ut_b, uu_b, up_b, g_b, sem):
        wid = lax.axis_index("s") * _NC + lax.axis_index("c")
        for ci in range(n_chunks):
            base = wid * rows_per_w + ci * _CHUNK
            pltpu.sync_copy(users_h.at[pl.ds(base, _CHUNK)], idx_v)
            pltpu.async_copy(ut_tab.at[idx_v], ut_b, sem).wait()
            pltpu.async_copy(u2_tab.at[idx_v], uu_b, sem).wait()

            @pl.loop(0, _CHUNK)
            def _(r):
                for h in range(D // 16):
                    sl = pl.ds(h * 16, 16)
                    up_b[r, sl] = (ut_b[r, sl] * 2.0 + uu_b[r, sl]
                                   + uu_b[r, pl.ds(D + h * 16, 16)])

            pltpu.sync_copy(up_b, up_out.at[pl.ds(base, _CHUNK)])
            pltpu.sync_copy(das_h.at[pl.ds(base, _CHUNK)], idx_v)
            pltpu.async_copy(da_h.at[idx_v], g_b, sem).wait()
            pltpu.sync_copy(g_b, da_out.at[pl.ds(base, _CHUNK)])
            pltpu.sync_copy(pos_h.at[pl.ds(base, _CHUNK)], idx_v)
            pltpu.async_copy(v_tab.at[idx_v], g_b, sem).wait()
            pltpu.sync_copy(g_b, i_out.at[pl.ds(base, _CHUNK)])
            for kn in range(NNEG):
                pltpu.sync_copy(neg_h.at[pl.ds(kn * B + base, _CHUNK)], idx_v)
                pltpu.async_copy(v_tab.at[idx_v], g_b, sem).wait()
                pltpu.sync_copy(g_b, j_out.at[pl.ds(kn * B + base, _CHUNK)])

    return k(users, pos, neg_t, das_c, u_true, u2, v, da_tab)


def _tc_loss(up, dag, ig, jg, wt, b2):
    B, D = up.shape
    NNEG = jg.shape[0]
    BLK = 1024
    grid = B // BLK

    def body(up_ref, da_ref, i_ref, j_ref, w_ref, b_ref, out_ref):
        u = jnp.dot(up_ref[...], w_ref[...],
                    preferred_element_type=jnp.float32)
        u = u + b_ref[...] + da_ref[...]
        dpos = u - i_ref[...] + _EPS
        dp = jnp.sqrt(jnp.sum(dpos * dpos, axis=1))
        acc = jnp.zeros((), jnp.float32)
        for kn in range(NNEG):
            dneg = u - j_ref[kn] + _EPS
            dn = jnp.sqrt(jnp.sum(dneg * dneg, axis=1))
            acc = acc + jnp.sum(jnp.maximum(dp - dn + 1.0, 0.0))

        @pl.when(pl.program_id(0) == 0)
        def _():
            out_ref[...] = jnp.zeros_like(out_ref)

        out_ref[...] += (acc * (1.0 / B)).reshape(1, 1)

    out = pl.pallas_call(
        body,
        grid=(grid,),
        in_specs=[
            pl.BlockSpec((BLK, D), lambda i: (i, 0)),
            pl.BlockSpec((BLK, D), lambda i: (i, 0)),
            pl.BlockSpec((BLK, D), lambda i: (i, 0)),
            pl.BlockSpec((NNEG, BLK, D), lambda i: (0, i, 0)),
            pl.BlockSpec((D, D), lambda i: (0, 0)),
            pl.BlockSpec((1, D), lambda i: (0, 0)),
        ],
        out_specs=pl.BlockSpec((1, 1), lambda i: (0, 0)),
        out_shape=jax.ShapeDtypeStruct((1, 1), jnp.float32),
    )(up, dag, ig, jg, wt, b2)
    return out[0, 0]


def kernel(phase, users, pos_job_ids, behavior_ids, das, neg_job_id_lists,
           U_true, U, V, da_tab, W, b):
    del phase, behavior_ids
    USER, BEHm1, D = U.shape
    DA = da_tab.shape[0] - 1
    das_c = jnp.clip(das, 0, DA).astype(jnp.int32)
    u2 = U.reshape(USER, BEHm1 * D)
    neg_t = neg_job_id_lists.T.reshape(-1)
    up, dag, ig, jg = _sc_gather(users, pos_job_ids, neg_t, das_c,
                                 U_true, u2, V, da_tab)
    NNEG = neg_job_id_lists.shape[1]
    jg = jg.reshape(NNEG, users.shape[0], -1)
    return _tc_loss(up, dag, ig, jg, W.T, b.reshape(1, -1))
